# baseline (speedup 1.0000x reference)
import functools

import numpy as np

import jax
import jax.numpy as jnp
from jax.experimental import pallas as pl
from jax.experimental.pallas import tpu as pltpu

_P = 10
_B = 1024

_F_L1 = 1
_F_RESET = 2
_F_DIAG = 4
_F_L2 = 8
_F_WRITE = 16
_F_INIT = 32
_F_SUP = 64


def _build_schedule(p):
    rows, cols, flg, oblk, xblk = [], [], [], [], []
    for i in range(p):
        for jj in range(p):
            c = (i + 1 + jj) % p
            f = _F_L1
            if jj == 0:
                f |= _F_RESET
            if c == i:
                f |= _F_DIAG
            if c <= i:
                f |= _F_L2
            if c == 0:
                f |= _F_INIT
            if i == 0:
                f |= _F_SUP
            if i == p - 1 and c == i:
                f |= _F_WRITE
            rows.append(i)
            cols.append(c)
            flg.append(f)
            oblk.append(p - 1)
            xblk.append(c if i == 0 else 0)
    for i in range(p - 1):
        for c in range(i + 1, p):
            f = _F_L2
            if c == p - 1:
                f |= _F_WRITE
            rows.append(i)
            cols.append(c)
            flg.append(f)
            oblk.append(i)
            xblk.append(0)
    to32 = lambda a: np.asarray(a, dtype=np.int32)
    return to32(rows), to32(cols), to32(flg), to32(oblk), to32(xblk)


def _gcn_body(b, last, rows_ref, cols_ref, flg_ref, oblk_ref, xblk_ref,
              adj_ref, x_ref, w1_ref, b1_ref, w2_ref, b2_ref,
              out_ref, sup_ref, h_ref, s2_ref, oacc_ref):
    p = _P
    t = pl.program_id(0)
    i = rows_ref[t]
    c = cols_ref[t]
    f = flg_ref[t]
    c_edge = c == p - 1
    i_edge = i == p - 1
    dot = functools.partial(jax.lax.dot, preferred_element_type=jnp.float32)

    def cpred(ce):
        return c_edge if ce else jnp.logical_not(c_edge)

    def ipred(ie):
        return i_edge if ie else jnp.logical_not(i_edge)

    for ce in (False, True):
        cw = last if ce else b

        @pl.when(((f & _F_SUP) != 0) & cpred(ce))
        def _sup(cw=cw):
            sup_ref[pl.ds(c * b, cw), :] = dot(x_ref[:cw, :], w1_ref[...])

    for ce in (False, True):
        cw = last if ce else b

        @pl.when(((f & _F_L1) != 0) & cpred(ce))
        def _l1(cw=cw):
            part = dot(adj_ref[:, :cw], sup_ref[pl.ds(c * b, cw), :])

            @pl.when((f & _F_RESET) != 0)
            def _set():
                h_ref[...] = part

            @pl.when((f & _F_RESET) == 0)
            def _acc():
                h_ref[...] = h_ref[...] + part

    for ie in (False, True):
        rw = last if ie else b

        @pl.when(((f & _F_DIAG) != 0) & ipred(ie))
        def _diag(rw=rw):
            hrow = jnp.maximum(h_ref[:rw, :] + b1_ref[...], 0.0)
            s2_ref[pl.ds(i * b, rw), :] = dot(hrow, w2_ref[...])

    for ce in (False, True):
        for ie in (False, True):
            cw = last if ce else b
            rw = last if ie else b

            @pl.when(((f & _F_L2) != 0) & cpred(ce) & ipred(ie))
            def _l2(cw=cw, rw=rw):
                contrib = dot(adj_ref[:rw, :cw],
                              s2_ref[pl.ds(c * b, cw), :])

                @pl.when((f & _F_INIT) != 0)
                def _set():
                    oacc_ref[pl.ds(i * b, rw), :] = contrib

                @pl.when((f & _F_INIT) == 0)
                def _acc():
                    oacc_ref[pl.ds(i * b, rw), :] = (
                        oacc_ref[pl.ds(i * b, rw), :] + contrib)

    for ie in (False, True):
        rw = last if ie else b

        @pl.when(((f & _F_WRITE) != 0) & ipred(ie))
        def _write(rw=rw):
            z = oacc_ref[pl.ds(i * b, rw), :] + b2_ref[...]
            m = jnp.max(z, axis=1, keepdims=True)
            lse = jnp.log(jnp.sum(jnp.exp(z - m), axis=1, keepdims=True)) + m
            out_ref[:rw, :] = z - lse


def kernel(x, adj, W1, b1, W2, b2):
    n, d_in = x.shape
    d_h = W1.shape[1]
    d_out = W2.shape[1]
    p = _P
    b = _B
    last = n - (p - 1) * b

    rows, cols, flg, oblk, xblk = _build_schedule(p)
    t_total = rows.shape[0]

    b1r = b1.reshape(1, d_h)
    b2r = b2.reshape(1, d_out)

    grid_spec = pltpu.PrefetchScalarGridSpec(
        num_scalar_prefetch=5,
        grid=(t_total,),
        in_specs=[
            pl.BlockSpec((b, b), lambda t, r, c, f, o, xb: (r[t], c[t])),
            pl.BlockSpec((b, d_in), lambda t, r, c, f, o, xb: (xb[t], 0)),
            pl.BlockSpec((d_in, d_h), lambda t, r, c, f, o, xb: (0, 0)),
            pl.BlockSpec((1, d_h), lambda t, r, c, f, o, xb: (0, 0)),
            pl.BlockSpec((d_h, d_out), lambda t, r, c, f, o, xb: (0, 0)),
            pl.BlockSpec((1, d_out), lambda t, r, c, f, o, xb: (0, 0)),
        ],
        out_specs=pl.BlockSpec((b, d_out), lambda t, r, c, f, o, xb: (o[t], 0)),
        scratch_shapes=[
            pltpu.VMEM((n, d_h), jnp.float32),
            pltpu.VMEM((b, d_h), jnp.float32),
            pltpu.VMEM((n, d_out), jnp.float32),
            pltpu.VMEM((n, d_out), jnp.float32),
        ],
    )

    return pl.pallas_call(
        functools.partial(_gcn_body, b, last),
        grid_spec=grid_spec,
        out_shape=jax.ShapeDtypeStruct((n, d_out), jnp.float32),
        compiler_params=pltpu.CompilerParams(
            dimension_semantics=("arbitrary",),
        ),
    )(rows, cols, flg, oblk, xblk, adj, x, W1, b1r, W2, b2r)


# speedup vs baseline: 1.3537x; 1.3537x over previous
import functools

import numpy as np

import jax
import jax.numpy as jnp
from jax.experimental import pallas as pl
from jax.experimental.pallas import tpu as pltpu

_P = 5
_B = 2048

_F_L1 = 1
_F_RESET = 2
_F_DIAG = 4
_F_L2 = 8
_F_WRITE = 16
_F_INIT = 32
_F_SUP = 64


def _build_schedule(p):
    rows, cols, flg, oblk, xblk = [], [], [], [], []
    for i in range(p):
        for jj in range(p):
            c = (i + 1 + jj) % p
            f = _F_L1
            if jj == 0:
                f |= _F_RESET
            if c == i:
                f |= _F_DIAG
            if c <= i:
                f |= _F_L2
            if c == 0:
                f |= _F_INIT
            if i == 0:
                f |= _F_SUP
            if i == p - 1 and c == i:
                f |= _F_WRITE
            rows.append(i)
            cols.append(c)
            flg.append(f)
            oblk.append(p - 1)
            xblk.append(c if i == 0 else 0)
    for i in range(p - 1):
        for c in range(i + 1, p):
            f = _F_L2
            if c == p - 1:
                f |= _F_WRITE
            rows.append(i)
            cols.append(c)
            flg.append(f)
            oblk.append(i)
            xblk.append(0)
    to32 = lambda a: np.asarray(a, dtype=np.int32)
    return to32(rows), to32(cols), to32(flg), to32(oblk), to32(xblk)


def _gcn_body(b, last, rows_ref, cols_ref, flg_ref, oblk_ref, xblk_ref,
              adj_ref, x_ref, w1_ref, b1_ref, w2_ref, b2_ref,
              out_ref, sup_ref, h_ref, s2_ref, oacc_ref):
    p = _P
    t = pl.program_id(0)
    i = rows_ref[t]
    c = cols_ref[t]
    f = flg_ref[t]
    c_edge = c == p - 1
    i_edge = i == p - 1
    dot = functools.partial(jax.lax.dot, preferred_element_type=jnp.float32)

    def cpred(ce):
        return c_edge if ce else jnp.logical_not(c_edge)

    def ipred(ie):
        return i_edge if ie else jnp.logical_not(i_edge)

    for ce in (False, True):
        cw = last if ce else b

        @pl.when(((f & _F_SUP) != 0) & cpred(ce))
        def _sup(cw=cw):
            sup_ref[pl.ds(c * b, cw), :] = dot(x_ref[:cw, :], w1_ref[...])

    for ce in (False, True):
        cw = last if ce else b

        @pl.when(((f & _F_L1) != 0) & cpred(ce))
        def _l1(cw=cw):
            part = dot(adj_ref[:, :cw], sup_ref[pl.ds(c * b, cw), :])

            @pl.when((f & _F_RESET) != 0)
            def _set():
                h_ref[...] = part

            @pl.when((f & _F_RESET) == 0)
            def _acc():
                h_ref[...] = h_ref[...] + part

    for ie in (False, True):
        rw = last if ie else b

        @pl.when(((f & _F_DIAG) != 0) & ipred(ie))
        def _diag(rw=rw):
            hrow = jnp.maximum(h_ref[:rw, :] + b1_ref[...], 0.0)
            s2_ref[pl.ds(i * b, rw), :] = dot(hrow, w2_ref[...])

    for ce in (False, True):
        for ie in (False, True):
            cw = last if ce else b
            rw = last if ie else b

            @pl.when(((f & _F_L2) != 0) & cpred(ce) & ipred(ie))
            def _l2(cw=cw, rw=rw):
                contrib = dot(adj_ref[:rw, :cw],
                              s2_ref[pl.ds(c * b, cw), :])

                @pl.when((f & _F_INIT) != 0)
                def _set():
                    oacc_ref[pl.ds(i * b, rw), :] = contrib

                @pl.when((f & _F_INIT) == 0)
                def _acc():
                    oacc_ref[pl.ds(i * b, rw), :] = (
                        oacc_ref[pl.ds(i * b, rw), :] + contrib)

    for ie in (False, True):
        rw = last if ie else b

        @pl.when(((f & _F_WRITE) != 0) & ipred(ie))
        def _write(rw=rw):
            z = oacc_ref[pl.ds(i * b, rw), :] + b2_ref[...]
            m = jnp.max(z, axis=1, keepdims=True)
            lse = jnp.log(jnp.sum(jnp.exp(z - m), axis=1, keepdims=True)) + m
            out_ref[:rw, :] = z - lse


def kernel(x, adj, W1, b1, W2, b2):
    n, d_in = x.shape
    d_h = W1.shape[1]
    d_out = W2.shape[1]
    p = _P
    b = _B
    last = n - (p - 1) * b

    rows, cols, flg, oblk, xblk = _build_schedule(p)
    t_total = rows.shape[0]

    b1r = b1.reshape(1, d_h)
    b2r = b2.reshape(1, d_out)

    grid_spec = pltpu.PrefetchScalarGridSpec(
        num_scalar_prefetch=5,
        grid=(t_total,),
        in_specs=[
            pl.BlockSpec((b, b), lambda t, r, c, f, o, xb: (r[t], c[t])),
            pl.BlockSpec((b, d_in), lambda t, r, c, f, o, xb: (xb[t], 0)),
            pl.BlockSpec((d_in, d_h), lambda t, r, c, f, o, xb: (0, 0)),
            pl.BlockSpec((1, d_h), lambda t, r, c, f, o, xb: (0, 0)),
            pl.BlockSpec((d_h, d_out), lambda t, r, c, f, o, xb: (0, 0)),
            pl.BlockSpec((1, d_out), lambda t, r, c, f, o, xb: (0, 0)),
        ],
        out_specs=pl.BlockSpec((b, d_out), lambda t, r, c, f, o, xb: (o[t], 0)),
        scratch_shapes=[
            pltpu.VMEM((n, d_h), jnp.float32),
            pltpu.VMEM((b, d_h), jnp.float32),
            pltpu.VMEM((n, d_out), jnp.float32),
            pltpu.VMEM((n, d_out), jnp.float32),
        ],
    )

    return pl.pallas_call(
        functools.partial(_gcn_body, b, last),
        grid_spec=grid_spec,
        out_shape=jax.ShapeDtypeStruct((n, d_out), jnp.float32),
        compiler_params=pltpu.CompilerParams(
            dimension_semantics=("arbitrary",),
        ),
    )(rows, cols, flg, oblk, xblk, adj, x, W1, b1r, W2, b2r)


# speedup vs baseline: 1.3551x; 1.0011x over previous
import functools

import numpy as np

import jax
import jax.numpy as jnp
from jax.experimental import pallas as pl
from jax.experimental.pallas import tpu as pltpu

_P = 5
_B = 2048
_H = _B // 2

_F_L1 = 1
_F_RESET = 2
_F_DIAG = 4
_F_L2 = 8
_F_WRITE = 16
_F_INIT = 32
_F_SUP = 64


def _build_schedule(p):
    rows, cols, flg, oblk, xblk = [], [], [], [], []
    for i in range(p):
        for jj in range(p):
            c = (i + 1 + jj) % p
            f = _F_L1
            if jj == 0:
                f |= _F_RESET
            if c == i:
                f |= _F_DIAG
            if c <= i:
                f |= _F_L2
            if c == 0:
                f |= _F_INIT
            if i == 0:
                f |= _F_SUP
            if i == p - 1 and c == i:
                f |= _F_WRITE
            rows.append(i)
            cols.append(c)
            flg.append(f)
            oblk.append(p - 1)
            xblk.append(c if i == 0 else 0)
    for i in range(p - 1):
        for c in range(i + 1, p):
            f = _F_L2
            if c == p - 1:
                f |= _F_WRITE
            rows.append(i)
            cols.append(c)
            flg.append(f)
            oblk.append(i)
            xblk.append(0)
    to32 = lambda a: np.asarray(a, dtype=np.int32)
    return to32(rows), to32(cols), to32(flg), to32(oblk), to32(xblk)


def _gcn_body(b, last, rows_ref, cols_ref, flg_ref, oblk_ref, xblk_ref,
              top_ref, bot_ref, x_ref, w1_ref, b1_ref, w2_ref, b2_ref,
              out_ref, sup_ref, h_ref, s2_ref, oacc_ref):
    p = _P
    hh = _H
    t = pl.program_id(0)
    i = rows_ref[t]
    c = cols_ref[t]
    f = flg_ref[t]
    c_edge = c == p - 1
    i_edge = i == p - 1
    last_b = last - hh
    dot = functools.partial(jax.lax.dot, preferred_element_type=jnp.float32)

    def cpred(ce):
        return c_edge if ce else jnp.logical_not(c_edge)

    def ipred(ie):
        return i_edge if ie else jnp.logical_not(i_edge)

    for ce in (False, True):
        cw = last if ce else b

        @pl.when(((f & _F_SUP) != 0) & cpred(ce))
        def _sup(cw=cw):
            sup_ref[pl.ds(c * b, cw), :] = dot(x_ref[:cw, :], w1_ref[...])

    for ce in (False, True):
        cw = last if ce else b

        @pl.when(((f & _F_L1) != 0) & cpred(ce))
        def _l1(cw=cw):
            sup_c = sup_ref[pl.ds(c * b, cw), :]
            part_t = dot(top_ref[:, :cw], sup_c)
            part_b = dot(bot_ref[:, :cw], sup_c)

            @pl.when((f & _F_RESET) != 0)
            def _set():
                h_ref[:hh, :] = part_t
                h_ref[hh:, :] = part_b

            @pl.when((f & _F_RESET) == 0)
            def _acc():
                h_ref[:hh, :] = h_ref[:hh, :] + part_t
                h_ref[hh:, :] = h_ref[hh:, :] + part_b

    for ie in (False, True):
        rw = last if ie else b

        @pl.when(((f & _F_DIAG) != 0) & ipred(ie))
        def _diag(rw=rw):
            hrow = jnp.maximum(h_ref[:rw, :] + b1_ref[...], 0.0)
            s2_ref[pl.ds(i * b, rw), :] = dot(hrow, w2_ref[...])

    for ce in (False, True):
        for ie in (False, True):
            cw = last if ce else b
            rwb = last_b if ie else hh

            @pl.when(((f & _F_L2) != 0) & cpred(ce) & ipred(ie))
            def _l2(cw=cw, rwb=rwb):
                s2_c = s2_ref[pl.ds(c * b, cw), :]
                con_t = dot(top_ref[:, :cw], s2_c)
                con_b = dot(bot_ref[:rwb, :cw], s2_c)

                @pl.when((f & _F_INIT) != 0)
                def _set():
                    oacc_ref[pl.ds(i * b, hh), :] = con_t
                    oacc_ref[pl.ds(i * b + hh, rwb), :] = con_b

                @pl.when((f & _F_INIT) == 0)
                def _acc():
                    oacc_ref[pl.ds(i * b, hh), :] = (
                        oacc_ref[pl.ds(i * b, hh), :] + con_t)
                    oacc_ref[pl.ds(i * b + hh, rwb), :] = (
                        oacc_ref[pl.ds(i * b + hh, rwb), :] + con_b)

    for ie in (False, True):
        rw = last if ie else b

        @pl.when(((f & _F_WRITE) != 0) & ipred(ie))
        def _write(rw=rw):
            z = oacc_ref[pl.ds(i * b, rw), :] + b2_ref[...]
            m = jnp.max(z, axis=1, keepdims=True)
            lse = jnp.log(jnp.sum(jnp.exp(z - m), axis=1, keepdims=True)) + m
            out_ref[:rw, :] = z - lse


def kernel(x, adj, W1, b1, W2, b2):
    n, d_in = x.shape
    d_h = W1.shape[1]
    d_out = W2.shape[1]
    p = _P
    b = _B
    hh = _H
    last = n - (p - 1) * b

    rows, cols, flg, oblk, xblk = _build_schedule(p)
    t_total = rows.shape[0]

    b1r = b1.reshape(1, d_h)
    b2r = b2.reshape(1, d_out)

    grid_spec = pltpu.PrefetchScalarGridSpec(
        num_scalar_prefetch=5,
        grid=(t_total,),
        in_specs=[
            pl.BlockSpec((hh, b), lambda t, r, c, f, o, xb: (2 * r[t], c[t])),
            pl.BlockSpec((hh, b),
                         lambda t, r, c, f, o, xb: (2 * r[t] + 1, c[t])),
            pl.BlockSpec((b, d_in), lambda t, r, c, f, o, xb: (xb[t], 0)),
            pl.BlockSpec((d_in, d_h), lambda t, r, c, f, o, xb: (0, 0)),
            pl.BlockSpec((1, d_h), lambda t, r, c, f, o, xb: (0, 0)),
            pl.BlockSpec((d_h, d_out), lambda t, r, c, f, o, xb: (0, 0)),
            pl.BlockSpec((1, d_out), lambda t, r, c, f, o, xb: (0, 0)),
        ],
        out_specs=pl.BlockSpec((b, d_out), lambda t, r, c, f, o, xb: (o[t], 0)),
        scratch_shapes=[
            pltpu.VMEM((n, d_h), jnp.float32),
            pltpu.VMEM((b, d_h), jnp.float32),
            pltpu.VMEM((n, d_out), jnp.float32),
            pltpu.VMEM((n, d_out), jnp.float32),
        ],
    )

    return pl.pallas_call(
        functools.partial(_gcn_body, b, last),
        grid_spec=grid_spec,
        out_shape=jax.ShapeDtypeStruct((n, d_out), jnp.float32),
        compiler_params=pltpu.CompilerParams(
            dimension_semantics=("arbitrary",),
        ),
    )(rows, cols, flg, oblk, xblk, adj, adj, x, W1, b1r, W2, b2r)


# speedup vs baseline: 1.3578x; 1.0020x over previous
import functools

import numpy as np

import jax
import jax.numpy as jnp
from jax.experimental import pallas as pl
from jax.experimental.pallas import tpu as pltpu

_P = 5
_B = 2048
_H = _B // 2

_F_L1 = 1
_F_RESET = 2
_F_DIAG = 4
_F_L2 = 8
_F_WRITE = 16
_F_INIT = 32
_F_SUP = 64
_F_HOLD = 128
_F_CONS = 256

_HELD = ((0, 1), (2, 3))


def _build_schedule(p):
    rows, cols, flg, oblk, xblk = [], [], [], [], []
    for i in range(p):
        for jj in range(p):
            c = (i + 1 + jj) % p
            f = _F_L1
            if jj == 0:
                f |= _F_RESET
            if c == i:
                f |= _F_DIAG
                if (i - 1, i) in _HELD:
                    f |= _F_CONS
            if c <= i:
                f |= _F_L2
            if c == 0:
                f |= _F_INIT
            if i == 0:
                f |= _F_SUP
            if (i, c) in _HELD:
                f |= _F_HOLD
            if i == p - 1 and c == i:
                f |= _F_WRITE
            rows.append(i)
            cols.append(c)
            flg.append(f)
            oblk.append(p - 1)
            xblk.append(c if i == 0 else 0)
    for i in range(p - 1):
        for c in range(i + 1, p):
            if (i, c) in _HELD:
                continue
            f = _F_L2
            if c == p - 1:
                f |= _F_WRITE
            rows.append(i)
            cols.append(c)
            flg.append(f)
            oblk.append(i)
            xblk.append(0)
    to32 = lambda a: np.asarray(a, dtype=np.int32)
    return to32(rows), to32(cols), to32(flg), to32(oblk), to32(xblk)


def _gcn_body(b, last, rows_ref, cols_ref, flg_ref, oblk_ref, xblk_ref,
              top_ref, bot_ref, x_ref, w1_ref, b1_ref, w2_ref, b2_ref,
              out_ref, sup_ref, h_ref, so_ref, hold_ref):
    p = _P
    hh = _H
    d = 64
    t = pl.program_id(0)
    i = rows_ref[t]
    c = cols_ref[t]
    f = flg_ref[t]
    c_edge = c == p - 1
    i_edge = i == p - 1
    last_b = last - hh
    dot = functools.partial(jax.lax.dot, preferred_element_type=jnp.float32)

    def cpred(ce):
        return c_edge if ce else jnp.logical_not(c_edge)

    def ipred(ie):
        return i_edge if ie else jnp.logical_not(i_edge)

    for ce in (False, True):
        cw = last if ce else b

        @pl.when(((f & _F_SUP) != 0) & cpred(ce))
        def _sup(cw=cw):
            sup_ref[pl.ds(c * b, cw), :] = dot(x_ref[:cw, :], w1_ref[...])

    for ce in (False, True):
        cw = last if ce else b

        @pl.when(((f & _F_L1) != 0) & cpred(ce))
        def _l1(cw=cw):
            sup_c = sup_ref[pl.ds(c * b, cw), :]
            part_t = dot(top_ref[:, :cw], sup_c)
            part_b = dot(bot_ref[:, :cw], sup_c)

            @pl.when((f & _F_RESET) != 0)
            def _set():
                h_ref[:hh, :] = part_t
                h_ref[hh:, :] = part_b

            @pl.when((f & _F_RESET) == 0)
            def _acc():
                h_ref[:hh, :] = h_ref[:hh, :] + part_t
                h_ref[hh:, :] = h_ref[hh:, :] + part_b

    @pl.when((f & _F_HOLD) != 0)
    def _hold():
        hold_ref[:hh, :] = top_ref[...].astype(jnp.bfloat16)
        hold_ref[hh:, :] = bot_ref[...].astype(jnp.bfloat16)

    for ie in (False, True):
        rw = last if ie else b

        @pl.when(((f & _F_DIAG) != 0) & ipred(ie))
        def _diag(rw=rw):
            hrow = jnp.maximum(h_ref[:rw, :] + b1_ref[...], 0.0)
            so_ref[pl.ds(i * b, rw), 0:d] = dot(hrow, w2_ref[...])

    @pl.when((f & _F_CONS) != 0)
    def _cons():
        s2c = so_ref[pl.ds(i * b, b), 0:d].astype(jnp.bfloat16)
        contrib = dot(hold_ref[...], s2c)
        tgt = pl.ds((i - 1) * b, b)
        so_ref[tgt, d:] = so_ref[tgt, d:] + contrib

    for ce in (False, True):
        for ie in (False, True):
            cw = last if ce else b
            rwb = last_b if ie else hh

            @pl.when(((f & _F_L2) != 0) & cpred(ce) & ipred(ie))
            def _l2(cw=cw, rwb=rwb):
                s2_c = so_ref[pl.ds(c * b, cw), 0:d]
                con_t = dot(top_ref[:, :cw], s2_c)
                con_b = dot(bot_ref[:rwb, :cw], s2_c)

                @pl.when((f & _F_INIT) != 0)
                def _set():
                    so_ref[pl.ds(i * b, hh), d:] = con_t
                    so_ref[pl.ds(i * b + hh, rwb), d:] = con_b

                @pl.when((f & _F_INIT) == 0)
                def _acc():
                    so_ref[pl.ds(i * b, hh), d:] = (
                        so_ref[pl.ds(i * b, hh), d:] + con_t)
                    so_ref[pl.ds(i * b + hh, rwb), d:] = (
                        so_ref[pl.ds(i * b + hh, rwb), d:] + con_b)

    for ie in (False, True):
        rw = last if ie else b

        @pl.when(((f & _F_WRITE) != 0) & ipred(ie))
        def _write(rw=rw):
            z = so_ref[pl.ds(i * b, rw), d:] + b2_ref[...]
            m = jnp.max(z, axis=1, keepdims=True)
            lse = jnp.log(jnp.sum(jnp.exp(z - m), axis=1, keepdims=True)) + m
            out_ref[:rw, :] = z - lse


def kernel(x, adj, W1, b1, W2, b2):
    n, d_in = x.shape
    d_h = W1.shape[1]
    d_out = W2.shape[1]
    p = _P
    b = _B
    last = n - (p - 1) * b

    rows, cols, flg, oblk, xblk = _build_schedule(p)
    t_total = rows.shape[0]

    b1r = b1.reshape(1, d_h)
    b2r = b2.reshape(1, d_out)

    grid_spec = pltpu.PrefetchScalarGridSpec(
        num_scalar_prefetch=5,
        grid=(t_total,),
        in_specs=[
            pl.BlockSpec((_H, b), lambda t, r, c, f, o, xb: (2 * r[t], c[t])),
            pl.BlockSpec((_H, b),
                         lambda t, r, c, f, o, xb: (2 * r[t] + 1, c[t])),
            pl.BlockSpec((b, d_in), lambda t, r, c, f, o, xb: (xb[t], 0)),
            pl.BlockSpec((d_in, d_h), lambda t, r, c, f, o, xb: (0, 0)),
            pl.BlockSpec((1, d_h), lambda t, r, c, f, o, xb: (0, 0)),
            pl.BlockSpec((d_h, d_out), lambda t, r, c, f, o, xb: (0, 0)),
            pl.BlockSpec((1, d_out), lambda t, r, c, f, o, xb: (0, 0)),
        ],
        out_specs=pl.BlockSpec((b, d_out), lambda t, r, c, f, o, xb: (o[t], 0)),
        scratch_shapes=[
            pltpu.VMEM((n, d_h), jnp.float32),
            pltpu.VMEM((b, d_h), jnp.float32),
            pltpu.VMEM((n, 2 * d_out), jnp.float32),
            pltpu.VMEM((b, b), jnp.bfloat16),
        ],
    )

    return pl.pallas_call(
        functools.partial(_gcn_body, b, last),
        grid_spec=grid_spec,
        out_shape=jax.ShapeDtypeStruct((n, d_out), jnp.float32),
        compiler_params=pltpu.CompilerParams(
            dimension_semantics=("arbitrary",),
            vmem_limit_bytes=63 * 1024 * 1024,
        ),
    )(rows, cols, flg, oblk, xblk, adj, adj, x, W1, b1r, W2, b2r)


# speedup vs baseline: 1.3675x; 1.0072x over previous
import functools

import numpy as np

import jax
import jax.numpy as jnp
from jax.experimental import pallas as pl
from jax.experimental.pallas import tpu as pltpu

_P = 5
_B = 2048
_H = _B // 2

_F_L1 = 1
_F_RESET = 2
_F_DIAG = 4
_F_L2 = 8
_F_WRITE = 16
_F_INIT = 32
_F_SUP = 64
_F_L2D = 128


def _build_schedule(p):
    rows, cols, flg, oblk, xblk = [], [], [], [], []
    for i in range(p):
        for jj in range(p):
            c = (i + 1 + jj) % p
            f = _F_L1
            if jj == 0:
                f |= _F_RESET
            if c == i:
                f |= _F_DIAG | _F_L2D
            if c < i:
                f |= _F_L2
            if c == 0:
                f |= _F_INIT
            if i == 0:
                f |= _F_SUP
            if i == p - 1 and c == i:
                f |= _F_WRITE
            rows.append(i)
            cols.append(c)
            flg.append(f)
            oblk.append(p - 1)
            xblk.append(c if i == 0 else 0)
    for i in range(p - 1):
        for c in range(i + 1, p):
            f = _F_L2
            if c == p - 1:
                f |= _F_WRITE
            rows.append(i)
            cols.append(c)
            flg.append(f)
            oblk.append(i)
            xblk.append(0)
    to32 = lambda a: np.asarray(a, dtype=np.int32)
    return to32(rows), to32(cols), to32(flg), to32(oblk), to32(xblk)


def _gcn_body(b, last, rows_ref, cols_ref, flg_ref, oblk_ref, xblk_ref,
              top_ref, bot_ref, x_ref, w1_ref, b1_ref, w2_ref, b2_ref,
              out_ref, rhs_ref, h_ref, oacc_ref):
    p = _P
    hh = _H
    dh = 128
    t = pl.program_id(0)
    i = rows_ref[t]
    c = cols_ref[t]
    f = flg_ref[t]
    c_edge = c == p - 1
    i_edge = i == p - 1
    last_b = last - hh
    dot = functools.partial(jax.lax.dot, preferred_element_type=jnp.float32)

    def cpred(ce):
        return c_edge if ce else jnp.logical_not(c_edge)

    def ipred(ie):
        return i_edge if ie else jnp.logical_not(i_edge)

    for ce in (False, True):
        cw = last if ce else b

        @pl.when(((f & _F_SUP) != 0) & cpred(ce))
        def _sup(cw=cw):
            rhs_ref[pl.ds(c * b, cw), 0:dh] = dot(x_ref[:cw, :], w1_ref[...])

    for ce in (False, True):
        cw = last if ce else b

        @pl.when(cpred(ce))
        def _main(cw=cw):
            rhs = rhs_ref[pl.ds(c * b, cw), :]
            res_t = dot(top_ref[:, :cw], rhs)
            res_b = dot(bot_ref[:, :cw], rhs)

            @pl.when((f & _F_RESET) != 0)
            def _l1_set():
                h_ref[:hh, :] = res_t[:, :dh]
                h_ref[hh:, :] = res_b[:, :dh]

            @pl.when(((f & _F_L1) != 0) & ((f & _F_RESET) == 0))
            def _l1_acc():
                h_ref[:hh, :] = h_ref[:hh, :] + res_t[:, :dh]
                h_ref[hh:, :] = h_ref[hh:, :] + res_b[:, :dh]

            for ie in (False, True):
                rwb = last_b if ie else hh

                @pl.when(((f & _F_L2) != 0) & ((f & _F_INIT) != 0)
                         & ipred(ie))
                def _l2_set(rwb=rwb):
                    oacc_ref[pl.ds(i * b, hh), :] = res_t[:, dh:]
                    oacc_ref[pl.ds(i * b + hh, rwb), :] = res_b[:rwb, dh:]

                @pl.when(((f & _F_L2) != 0) & ((f & _F_INIT) == 0)
                         & ipred(ie))
                def _l2_acc(rwb=rwb):
                    oacc_ref[pl.ds(i * b, hh), :] = (
                        oacc_ref[pl.ds(i * b, hh), :] + res_t[:, dh:])
                    oacc_ref[pl.ds(i * b + hh, rwb), :] = (
                        oacc_ref[pl.ds(i * b + hh, rwb), :]
                        + res_b[:rwb, dh:])

    for ie in (False, True):
        rw = last if ie else b
        rwb = last_b if ie else hh

        @pl.when(((f & _F_DIAG) != 0) & ipred(ie))
        def _diag(rw=rw):
            hrow = jnp.maximum(h_ref[:rw, :] + b1_ref[...], 0.0)
            rhs_ref[pl.ds(i * b, rw), dh:] = dot(hrow, w2_ref[...])

        @pl.when(((f & _F_L2D) != 0) & ((f & _F_INIT) != 0) & ipred(ie))
        def _l2d_set(rw=rw, rwb=rwb):
            s2c = rhs_ref[pl.ds(i * b, rw), dh:]
            oacc_ref[pl.ds(i * b, hh), :] = dot(top_ref[:, :rw], s2c)
            oacc_ref[pl.ds(i * b + hh, rwb), :] = dot(
                bot_ref[:rwb, :rw], s2c)

        @pl.when(((f & _F_L2D) != 0) & ((f & _F_INIT) == 0) & ipred(ie))
        def _l2d_acc(rw=rw, rwb=rwb):
            s2c = rhs_ref[pl.ds(i * b, rw), dh:]
            oacc_ref[pl.ds(i * b, hh), :] = (
                oacc_ref[pl.ds(i * b, hh), :] + dot(top_ref[:, :rw], s2c))
            oacc_ref[pl.ds(i * b + hh, rwb), :] = (
                oacc_ref[pl.ds(i * b + hh, rwb), :]
                + dot(bot_ref[:rwb, :rw], s2c))

    for ie in (False, True):
        rw = last if ie else b

        @pl.when(((f & _F_WRITE) != 0) & ipred(ie))
        def _write(rw=rw):
            z = oacc_ref[pl.ds(i * b, rw), :] + b2_ref[...]
            m = jnp.max(z, axis=1, keepdims=True)
            lse = jnp.log(jnp.sum(jnp.exp(z - m), axis=1, keepdims=True)) + m
            out_ref[:rw, :] = z - lse


def kernel(x, adj, W1, b1, W2, b2):
    n, d_in = x.shape
    d_h = W1.shape[1]
    d_out = W2.shape[1]
    p = _P
    b = _B
    last = n - (p - 1) * b

    rows, cols, flg, oblk, xblk = _build_schedule(p)
    t_total = rows.shape[0]

    b1r = b1.reshape(1, d_h)
    b2r = b2.reshape(1, d_out)

    grid_spec = pltpu.PrefetchScalarGridSpec(
        num_scalar_prefetch=5,
        grid=(t_total,),
        in_specs=[
            pl.BlockSpec((_H, b), lambda t, r, c, f, o, xb: (2 * r[t], c[t])),
            pl.BlockSpec((_H, b),
                         lambda t, r, c, f, o, xb: (2 * r[t] + 1, c[t])),
            pl.BlockSpec((b, d_in), lambda t, r, c, f, o, xb: (xb[t], 0)),
            pl.BlockSpec((d_in, d_h), lambda t, r, c, f, o, xb: (0, 0)),
            pl.BlockSpec((1, d_h), lambda t, r, c, f, o, xb: (0, 0)),
            pl.BlockSpec((d_h, d_out), lambda t, r, c, f, o, xb: (0, 0)),
            pl.BlockSpec((1, d_out), lambda t, r, c, f, o, xb: (0, 0)),
        ],
        out_specs=pl.BlockSpec((b, d_out), lambda t, r, c, f, o, xb: (o[t], 0)),
        scratch_shapes=[
            pltpu.VMEM((n, d_h + d_out), jnp.float32),
            pltpu.VMEM((b, d_h), jnp.float32),
            pltpu.VMEM((n, d_out), jnp.float32),
        ],
    )

    return pl.pallas_call(
        functools.partial(_gcn_body, b, last),
        grid_spec=grid_spec,
        out_shape=jax.ShapeDtypeStruct((n, d_out), jnp.float32),
        compiler_params=pltpu.CompilerParams(
            dimension_semantics=("arbitrary",),
            vmem_limit_bytes=63 * 1024 * 1024,
        ),
    )(rows, cols, flg, oblk, xblk, adj, adj, x, W1, b1r, W2, b2r)


# speedup vs baseline: 1.3952x; 1.0203x over previous
import functools

import numpy as np

import jax
import jax.numpy as jnp
from jax.experimental import pallas as pl
from jax.experimental.pallas import tpu as pltpu

_P = 5
_B = 2048
_H = _B // 2

_F_L1 = 1
_F_RESET = 2
_F_DIAG = 4
_F_L2 = 8
_F_WRITE = 16
_F_INIT = 32
_F_SUP = 64
_F_L2D = 128
_F_HOLD = 256
_F_CONS = 512

_HELD = ((0, 1), (2, 3))


def _build_schedule(p):
    rows, cols, flg, oblk, xblk = [], [], [], [], []
    for i in range(p):
        for jj in range(p):
            c = (i + 1 + jj) % p
            f = _F_L1
            if jj == 0:
                f |= _F_RESET
            if c == i:
                f |= _F_DIAG | _F_L2D
                if (i - 1, i) in _HELD:
                    f |= _F_CONS
            if c < i:
                f |= _F_L2
            if c == 0:
                f |= _F_INIT
            if i == 0:
                f |= _F_SUP
            if (i, c) in _HELD:
                f |= _F_HOLD
            if i == p - 1 and c == i:
                f |= _F_WRITE
            rows.append(i)
            cols.append(c)
            flg.append(f)
            oblk.append(p - 1)
            xblk.append(c if i == 0 else 0)
    for i in range(p - 1):
        for c in range(i + 1, p):
            if (i, c) in _HELD:
                continue
            f = _F_L2
            if c == p - 1:
                f |= _F_WRITE
            rows.append(i)
            cols.append(c)
            flg.append(f)
            oblk.append(i)
            xblk.append(0)
    to32 = lambda a: np.asarray(a, dtype=np.int32)
    return to32(rows), to32(cols), to32(flg), to32(oblk), to32(xblk)


def _gcn_body(b, last, rows_ref, cols_ref, flg_ref, oblk_ref, xblk_ref,
              top_ref, bot_ref, x_ref, w1_ref, b1_ref, w2_ref, b2_ref,
              out_ref, rhs_ref, h_ref, oacc_ref, hold_ref):
    p = _P
    hh = _H
    dh = 128
    t = pl.program_id(0)
    i = rows_ref[t]
    c = cols_ref[t]
    f = flg_ref[t]
    c_edge = c == p - 1
    i_edge = i == p - 1
    last_b = last - hh
    dot = functools.partial(jax.lax.dot, preferred_element_type=jnp.float32)

    def cpred(ce):
        return c_edge if ce else jnp.logical_not(c_edge)

    def ipred(ie):
        return i_edge if ie else jnp.logical_not(i_edge)

    for ce in (False, True):
        cw = last if ce else b

        @pl.when(((f & _F_SUP) != 0) & cpred(ce))
        def _sup(cw=cw):
            rhs_ref[pl.ds(c * b, cw), 0:dh] = dot(x_ref[:cw, :], w1_ref[...])

    for ce in (False, True):
        cw = last if ce else b

        @pl.when(cpred(ce))
        def _main(cw=cw):
            rhs = rhs_ref[pl.ds(c * b, cw), :]
            res_t = dot(top_ref[:, :cw], rhs)
            res_b = dot(bot_ref[:, :cw], rhs)

            @pl.when((f & _F_RESET) != 0)
            def _l1_set():
                h_ref[:hh, :] = res_t[:, :dh]
                h_ref[hh:, :] = res_b[:, :dh]

            @pl.when(((f & _F_L1) != 0) & ((f & _F_RESET) == 0))
            def _l1_acc():
                h_ref[:hh, :] = h_ref[:hh, :] + res_t[:, :dh]
                h_ref[hh:, :] = h_ref[hh:, :] + res_b[:, :dh]

            for ie in (False, True):
                rwb = last_b if ie else hh

                @pl.when(((f & _F_L2) != 0) & ((f & _F_INIT) != 0)
                         & ipred(ie))
                def _l2_set(rwb=rwb):
                    oacc_ref[pl.ds(i * b, hh), :] = res_t[:, dh:]
                    oacc_ref[pl.ds(i * b + hh, rwb), :] = res_b[:rwb, dh:]

                @pl.when(((f & _F_L2) != 0) & ((f & _F_INIT) == 0)
                         & ipred(ie))
                def _l2_acc(rwb=rwb):
                    oacc_ref[pl.ds(i * b, hh), :] = (
                        oacc_ref[pl.ds(i * b, hh), :] + res_t[:, dh:])
                    oacc_ref[pl.ds(i * b + hh, rwb), :] = (
                        oacc_ref[pl.ds(i * b + hh, rwb), :]
                        + res_b[:rwb, dh:])

    @pl.when((f & _F_HOLD) != 0)
    def _hold():
        hold_ref[:hh, :] = top_ref[...].astype(jnp.bfloat16)
        hold_ref[hh:, :] = bot_ref[...].astype(jnp.bfloat16)

    for ie in (False, True):
        rw = last if ie else b
        rwb = last_b if ie else hh

        @pl.when(((f & _F_DIAG) != 0) & ipred(ie))
        def _diag(rw=rw):
            hrow = jnp.maximum(h_ref[:rw, :] + b1_ref[...], 0.0)
            rhs_ref[pl.ds(i * b, rw), dh:] = dot(hrow, w2_ref[...])

        @pl.when(((f & _F_L2D) != 0) & ((f & _F_INIT) != 0) & ipred(ie))
        def _l2d_set(rw=rw, rwb=rwb):
            s2c = rhs_ref[pl.ds(i * b, rw), dh:]
            oacc_ref[pl.ds(i * b, hh), :] = dot(top_ref[:, :rw], s2c)
            oacc_ref[pl.ds(i * b + hh, rwb), :] = dot(
                bot_ref[:rwb, :rw], s2c)

        @pl.when(((f & _F_L2D) != 0) & ((f & _F_INIT) == 0) & ipred(ie))
        def _l2d_acc(rw=rw, rwb=rwb):
            s2c = rhs_ref[pl.ds(i * b, rw), dh:]
            oacc_ref[pl.ds(i * b, hh), :] = (
                oacc_ref[pl.ds(i * b, hh), :] + dot(top_ref[:, :rw], s2c))
            oacc_ref[pl.ds(i * b + hh, rwb), :] = (
                oacc_ref[pl.ds(i * b + hh, rwb), :]
                + dot(bot_ref[:rwb, :rw], s2c))

    @pl.when((f & _F_CONS) != 0)
    def _cons():
        s2c = rhs_ref[pl.ds(i * b, b), dh:].astype(jnp.bfloat16)
        contrib = dot(hold_ref[...], s2c)
        tgt = pl.ds((i - 1) * b, b)
        oacc_ref[tgt, :] = oacc_ref[tgt, :] + contrib

    for ie in (False, True):
        rw = last if ie else b

        @pl.when(((f & _F_WRITE) != 0) & ipred(ie))
        def _write(rw=rw):
            z = oacc_ref[pl.ds(i * b, rw), :] + b2_ref[...]
            m = jnp.max(z, axis=1, keepdims=True)
            lse = jnp.log(jnp.sum(jnp.exp(z - m), axis=1, keepdims=True)) + m
            out_ref[:rw, :] = z - lse


def kernel(x, adj, W1, b1, W2, b2):
    n, d_in = x.shape
    d_h = W1.shape[1]
    d_out = W2.shape[1]
    p = _P
    b = _B
    last = n - (p - 1) * b

    rows, cols, flg, oblk, xblk = _build_schedule(p)
    t_total = rows.shape[0]

    b1r = b1.reshape(1, d_h)
    b2r = b2.reshape(1, d_out)

    grid_spec = pltpu.PrefetchScalarGridSpec(
        num_scalar_prefetch=5,
        grid=(t_total,),
        in_specs=[
            pl.BlockSpec((_H, b), lambda t, r, c, f, o, xb: (2 * r[t], c[t])),
            pl.BlockSpec((_H, b),
                         lambda t, r, c, f, o, xb: (2 * r[t] + 1, c[t])),
            pl.BlockSpec((b, d_in), lambda t, r, c, f, o, xb: (xb[t], 0)),
            pl.BlockSpec((d_in, d_h), lambda t, r, c, f, o, xb: (0, 0)),
            pl.BlockSpec((1, d_h), lambda t, r, c, f, o, xb: (0, 0)),
            pl.BlockSpec((d_h, d_out), lambda t, r, c, f, o, xb: (0, 0)),
            pl.BlockSpec((1, d_out), lambda t, r, c, f, o, xb: (0, 0)),
        ],
        out_specs=pl.BlockSpec((b, d_out), lambda t, r, c, f, o, xb: (o[t], 0)),
        scratch_shapes=[
            pltpu.VMEM((n, d_h + d_out), jnp.float32),
            pltpu.VMEM((b, d_h), jnp.float32),
            pltpu.VMEM((n, d_out), jnp.float32),
            pltpu.VMEM((b, b), jnp.bfloat16),
        ],
    )

    return pl.pallas_call(
        functools.partial(_gcn_body, b, last),
        grid_spec=grid_spec,
        out_shape=jax.ShapeDtypeStruct((n, d_out), jnp.float32),
        compiler_params=pltpu.CompilerParams(
            dimension_semantics=("arbitrary",),
            vmem_limit_bytes=63 * 1024 * 1024,
        ),
    )(rows, cols, flg, oblk, xblk, adj, adj, x, W1, b1r, W2, b2r)
